# baseline (speedup 1.0000x reference)
import functools

import jax
import jax.numpy as jnp
from jax import lax
from jax.experimental import pallas as pl
from jax.experimental.pallas import tpu as pltpu


def _pick_tile(dim, target):
    if dim <= target:
        return dim
    for t in range(target, 0, -1):
        if dim % t == 0:
            return t
    return dim


def _emb_gather_kernel(ids_ref, tab_ref, out_ref, *, unroll):
    rows = out_ref.shape[0]

    def outer(cc, carry):
        base = cc * unroll
        for u in range(unroll):
            idx = ids_ref[0, 0, base + u]
            out_ref[pl.ds(base + u, 1), 0, :] = tab_ref[pl.ds(idx, 1), 0, :]
        return carry

    lax.fori_loop(0, rows // unroll, outer, 0)


def _embedding_lookup(table, ids):
    T, B = ids.shape
    ntoken, ninp = table.shape
    N = T * B
    tab32 = table.astype(jnp.float32).reshape(ntoken, 1, ninp)

    rows = 512 if N % 512 == 0 else _pick_tile(N, 512)
    G = N // rows
    unroll = 64 if rows % 64 == 0 else rows
    ids3d = ids.reshape(G, 1, rows)

    out = pl.pallas_call(
        functools.partial(_emb_gather_kernel, unroll=unroll),
        out_shape=jax.ShapeDtypeStruct((N, 1, ninp), jnp.float32),
        grid=(G,),
        in_specs=[
            pl.BlockSpec((1, 1, rows), lambda i: (i, 0, 0),
                         memory_space=pltpu.SMEM),
            pl.BlockSpec((ntoken, 1, ninp), lambda i: (0, 0, 0)),
        ],
        out_specs=pl.BlockSpec((rows, 1, ninp), lambda i: (i, 0, 0)),
        compiler_params=pltpu.CompilerParams(
            dimension_semantics=("parallel",),
            vmem_limit_bytes=int(ntoken * ninp * 4 * 2 + 16 * 1024 * 1024),
        ),
    )(ids3d, tab32)
    return out.astype(jnp.bfloat16).reshape(T, B, ninp)


def _sigmoid(x):
    return 0.5 * jnp.tanh(0.5 * x) + 0.5


def _fused_lstm_kernel(x_ref, h0_ref, c0_ref,
                       wi0, wi1, wi2, wi3, wh0, wh1, wh2, wh3, b_ref,
                       y_ref, hT_ref, cT_ref,
                       yb_in, yb_mid, gxa, gxb, wih_s, whh_s, wsem,
                       *, tc, tb, H, nlayers, nchunks):
    bi = pl.program_id(0)
    lp = pl.program_id(1)
    s = pl.program_id(2)

    @pl.when((lp == 0) & (s == 0))
    def _():
        cps = []
        for i, (wi, wh) in enumerate(((wi0, wh0), (wi1, wh1),
                                      (wi2, wh2), (wi3, wh3))):
            cps.append(pltpu.make_async_copy(wi, wih_s.at[i], wsem))
            cps.append(pltpu.make_async_copy(wh, whh_s.at[i], wsem))
        cps.append(pltpu.make_async_copy(
            h0_ref.at[:, pl.ds(bi * tb, tb)], hT_ref, wsem))
        cps.append(pltpu.make_async_copy(
            c0_ref.at[:, pl.ds(bi * tb, tb)], cT_ref, wsem))
        for cp in cps:
            cp.start()
        for cp in cps:
            cp.wait()

    nc = nchunks
    la = 2 * lp
    lb = 2 * lp + 1

    @pl.when((lp == 0) & (s < nc))
    def _():
        yb_in[pl.ds(s * tc, tc)] = x_ref[...]

    ca = jnp.minimum(s, nc - 1)
    cb = jnp.maximum(s - 1, 0)
    a_valid = s < nc
    b_valid = s > 0
    wa = jnp.where(a_valid, ca, 0)
    wb = jnp.where(b_valid, cb, 0)

    xa = yb_in[pl.ds(ca * tc, tc)].reshape(tc * tb, H)
    gxa[...] = (jnp.dot(xa, wih_s[la], preferred_element_type=jnp.float32)
                + b_ref[pl.ds(la, 1)]).astype(jnp.bfloat16)
    xb = yb_mid[pl.ds(cb * tc, tc)].reshape(tc * tb, H)
    gxb[...] = (jnp.dot(xb, wih_s[lb], preferred_element_type=jnp.float32)
                + b_ref[pl.ds(lb, 1)]).astype(jnp.bfloat16)

    hbA = hT_ref[pl.ds(la, 1)].reshape(tb, H).astype(jnp.bfloat16)
    csA = cT_ref[pl.ds(la, 1)].reshape(tb, H)
    hbB = hT_ref[pl.ds(lb, 1)].reshape(tb, H).astype(jnp.bfloat16)
    csB = cT_ref[pl.ds(lb, 1)].reshape(tb, H)

    for t in range(tc):
        row = pl.ds(t * tb, tb)
        iA = _sigmoid(gxa[row, 0 * H:1 * H] + jnp.dot(
            hbA, whh_s[la, :, 0 * H:1 * H],
            preferred_element_type=jnp.float32))
        fA = _sigmoid(gxa[row, 1 * H:2 * H] + jnp.dot(
            hbA, whh_s[la, :, 1 * H:2 * H],
            preferred_element_type=jnp.float32))
        gA = jnp.tanh(gxa[row, 2 * H:3 * H] + jnp.dot(
            hbA, whh_s[la, :, 2 * H:3 * H],
            preferred_element_type=jnp.float32))
        oA = _sigmoid(gxa[row, 3 * H:4 * H] + jnp.dot(
            hbA, whh_s[la, :, 3 * H:4 * H],
            preferred_element_type=jnp.float32))
        csA = fA * csA + iA * gA
        hbA = (oA * jnp.tanh(csA)).astype(jnp.bfloat16)
        yb_mid[pl.ds(wa * tc + t, 1)] = hbA.reshape(1, tb, H)

        iB = _sigmoid(gxb[row, 0 * H:1 * H] + jnp.dot(
            hbB, whh_s[lb, :, 0 * H:1 * H],
            preferred_element_type=jnp.float32))
        fB = _sigmoid(gxb[row, 1 * H:2 * H] + jnp.dot(
            hbB, whh_s[lb, :, 1 * H:2 * H],
            preferred_element_type=jnp.float32))
        gB = jnp.tanh(gxb[row, 2 * H:3 * H] + jnp.dot(
            hbB, whh_s[lb, :, 2 * H:3 * H],
            preferred_element_type=jnp.float32))
        oB = _sigmoid(gxb[row, 3 * H:4 * H] + jnp.dot(
            hbB, whh_s[lb, :, 3 * H:4 * H],
            preferred_element_type=jnp.float32))
        csB = fB * csB + iB * gB
        hbB = (oB * jnp.tanh(csB)).astype(jnp.bfloat16)
        yb_in[pl.ds(wb * tc + t, 1)] = hbB.reshape(1, tb, H)

    @pl.when(a_valid)
    def _():
        hT_ref[pl.ds(la, 1)] = hbA.astype(jnp.float32).reshape(1, tb, H)
        cT_ref[pl.ds(la, 1)] = csA.reshape(1, tb, H)

    @pl.when(b_valid)
    def _():
        hT_ref[pl.ds(lb, 1)] = hbB.astype(jnp.float32).reshape(1, tb, H)
        cT_ref[pl.ds(lb, 1)] = csB.reshape(1, tb, H)

    @pl.when((lp == nlayers // 2 - 1) & b_valid)
    def _():
        y_ref[...] = yb_in[pl.ds(cb * tc, tc)].astype(jnp.float32)


def _fused_lstm(x, h0, c0, wih, whh, bcat):
    T, B, H = x.shape
    L = len(wih)
    tc = _pick_tile(T, 8)
    tb = B // 2 if (B >= 16 and B % 16 == 0) else B
    nb = B // tb
    nchunks = T // tc
    nc = nchunks
    kern = functools.partial(_fused_lstm_kernel, tc=tc, tb=tb, H=H,
                             nlayers=L, nchunks=nchunks)
    est = (2 * tc * tb * H * 2
           + 2 * L * H * 4 * H * 2
           + 2 * tc * tb * H * 4
           + 4 * L * tb * H * 4
           + T * tb * H * 2
           + tc * tb * 4 * H * 4
           + 12 * 1024 * 1024)
    any_spec = pl.BlockSpec(memory_space=pl.ANY)
    y, hT, cT = pl.pallas_call(
        kern,
        out_shape=(
            jax.ShapeDtypeStruct((T, B, H), jnp.float32),
            jax.ShapeDtypeStruct((L, B, H), jnp.float32),
            jax.ShapeDtypeStruct((L, B, H), jnp.float32),
        ),
        grid=(nb, L // 2, nchunks + 1),
        in_specs=[
            pl.BlockSpec((tc, tb, H),
                         lambda bi, lp, s: (jnp.where(
                             lp == 0, jnp.minimum(s, nc - 1), 0), bi, 0)),
            any_spec, any_spec,
            any_spec, any_spec, any_spec, any_spec,
            any_spec, any_spec, any_spec, any_spec,
            pl.BlockSpec((L, 4 * H), lambda bi, lp, s: (0, 0)),
        ],
        out_specs=[
            pl.BlockSpec((tc, tb, H),
                         lambda bi, lp, s: (jnp.where(
                             (lp == L // 2 - 1) & (s > 0),
                             jnp.maximum(s - 1, 0), 0), bi, 0)),
            pl.BlockSpec((L, tb, H), lambda bi, lp, s: (0, bi, 0)),
            pl.BlockSpec((L, tb, H), lambda bi, lp, s: (0, bi, 0)),
        ],
        scratch_shapes=[
            pltpu.VMEM((T, tb, H), jnp.bfloat16),
            pltpu.VMEM((T, tb, H), jnp.bfloat16),
            pltpu.VMEM((tc * tb, 4 * H), jnp.bfloat16),
            pltpu.VMEM((tc * tb, 4 * H), jnp.bfloat16),
            pltpu.VMEM((L, H, 4 * H), jnp.bfloat16),
            pltpu.VMEM((L, H, 4 * H), jnp.bfloat16),
            pltpu.SemaphoreType.DMA,
        ],
        compiler_params=pltpu.CompilerParams(
            dimension_semantics=("parallel", "arbitrary", "arbitrary"),
            vmem_limit_bytes=int(min(est, 56 * 1024 * 1024)),
        ),
    )(x, h0, c0, *wih, *whh, bcat)
    return y, hT, cT


def kernel(emb, input_ids, h0, c0,
           w_ih_0, w_hh_0, b_0,
           w_ih_1, w_hh_1, b_1,
           w_ih_2, w_hh_2, b_2,
           w_ih_3, w_hh_3, b_3):
    x = _embedding_lookup(emb, input_ids)
    bcat = jnp.concatenate([b_0, b_1, b_2, b_3], axis=0)
    y, hT, cT = _fused_lstm(x, h0, c0,
                            (w_ih_0, w_ih_1, w_ih_2, w_ih_3),
                            (w_hh_0, w_hh_1, w_hh_2, w_hh_3), bcat)
    return y, (hT, cT)
